# baseline (speedup 1.0000x reference)
import functools

import jax
import jax.numpy as jnp
from jax import lax
from jax.experimental import pallas as pl
from jax.experimental.pallas import tpu as pltpu
from jax.experimental.pallas import tpu_sc as plsc

N_TOKENS = 4096
D_IN = 1024
D_OUT = 1024
N_EXPERTS = 8
H = 1024

T = 128
P = N_TOKENS + N_EXPERTS * T
NT = P // T


def _gating_body(x_ref, w1_ref, b1_ref, w2_ref, b2_ref, w3_ref, b3_ref,
                 logits_ref, ids_ref):
    h = jnp.maximum(jnp.dot(x_ref[...], w1_ref[...],
                            preferred_element_type=jnp.float32) + b1_ref[...], 0.0)
    h = jnp.maximum(jnp.dot(h, w2_ref[...],
                            preferred_element_type=jnp.float32) + b2_ref[...], 0.0)
    lg = jnp.dot(h, w3_ref[...], preferred_element_type=jnp.float32) + b3_ref[...]
    logits_ref[...] = lg
    m = jnp.max(lg, axis=1, keepdims=True)
    cols = lax.broadcasted_iota(jnp.int32, lg.shape, 1)
    first_max = jnp.min(jnp.where(lg == m, cols, N_EXPERTS), axis=1, keepdims=True)
    ids_ref[...] = first_max.astype(jnp.int32)


def _gating(x, gW1, gb1, gW2, gb2, gW3, gb3):
    logits, ids = pl.pallas_call(
        _gating_body,
        out_shape=(
            jax.ShapeDtypeStruct((N_TOKENS, N_EXPERTS), jnp.float32),
            jax.ShapeDtypeStruct((N_TOKENS, 1), jnp.int32),
        ),
    )(x, gW1, gb1.reshape(1, -1), gW2, gb2.reshape(1, -1), gW3,
      gb3.reshape(1, -1))
    return logits, ids.reshape(N_TOKENS)


@functools.lru_cache(maxsize=None)
def _make_row_gather(n_rows_out, d):
    info = plsc.get_sparse_core_info()
    nc, ns = info.num_cores, info.num_subcores
    nw = nc * ns
    per_w = n_rows_out // nw
    ch = 32
    n_ch = per_w // ch
    mesh = plsc.VectorSubcoreMesh(core_axis_name="c", subcore_axis_name="s")

    @functools.partial(
        pl.kernel,
        mesh=mesh,
        out_type=jax.ShapeDtypeStruct((n_rows_out, d), jnp.float32),
        scratch_types=[
            pltpu.VMEM((n_ch, ch), jnp.int32),
            pltpu.VMEM((ch, d), jnp.float32),
            pltpu.VMEM((ch, d), jnp.float32),
            pltpu.SemaphoreType.DMA,
            pltpu.SemaphoreType.DMA,
            pltpu.SemaphoreType.DMA,
            pltpu.SemaphoreType.DMA,
        ],
    )
    def gather(table_hbm, idx_hbm, out_hbm, idx_v, buf0, buf1,
               gsem0, gsem1, osem0, osem1):
        wid = lax.axis_index("s") * nc + lax.axis_index("c")
        pltpu.sync_copy(idx_hbm.at[wid], idx_v)
        base = wid * per_w
        bufs, gsems, osems = (buf0, buf1), (gsem0, gsem1), (osem0, osem1)

        def start_gather(c):
            return pltpu.async_copy(table_hbm.at[idx_v.at[c]], bufs[c % 2],
                                    gsems[c % 2])

        def start_out(c):
            return pltpu.async_copy(bufs[c % 2],
                                    out_hbm.at[pl.ds(base + c * ch, ch)],
                                    osems[c % 2])

        cp = start_gather(0)
        outcp = [None, None]
        for c in range(n_ch):
            cp.wait()
            outcp[c % 2] = start_out(c)
            if c + 1 < n_ch:
                if outcp[(c + 1) % 2] is not None:
                    outcp[(c + 1) % 2].wait()
                cp = start_gather(c + 1)
        outcp[(n_ch - 1) % 2].wait()
        if n_ch >= 2:
            outcp[n_ch % 2].wait()

    def run(table, idx):
        return gather(table, idx.reshape(nw, n_ch, ch))

    return run


def _mlp_body(te_ref, xs_ref, w1_ref, b1_ref, w2_ref, b2_ref, w3_ref, b3_ref,
              o_ref):
    del te_ref
    h = jnp.maximum(jnp.dot(xs_ref[...], w1_ref[0],
                            preferred_element_type=jnp.float32) + b1_ref[0], 0.0)
    h = jnp.maximum(jnp.dot(h, w2_ref[0],
                            preferred_element_type=jnp.float32) + b2_ref[0], 0.0)
    o_ref[...] = jnp.dot(h, w3_ref[0],
                         preferred_element_type=jnp.float32) + b3_ref[0]


def _expert_mlp(tile_expert, x_sorted, eW1, eb1, eW2, eb2, eW3, eb3):
    grid_spec = pltpu.PrefetchScalarGridSpec(
        num_scalar_prefetch=1,
        grid=(NT,),
        in_specs=[
            pl.BlockSpec((T, D_IN), lambda t, te: (t, 0)),
            pl.BlockSpec((1, D_IN, H), lambda t, te: (te[t], 0, 0)),
            pl.BlockSpec((1, 1, H), lambda t, te: (te[t], 0, 0)),
            pl.BlockSpec((1, H, H), lambda t, te: (te[t], 0, 0)),
            pl.BlockSpec((1, 1, H), lambda t, te: (te[t], 0, 0)),
            pl.BlockSpec((1, H, D_OUT), lambda t, te: (te[t], 0, 0)),
            pl.BlockSpec((1, 1, D_OUT), lambda t, te: (te[t], 0, 0)),
        ],
        out_specs=pl.BlockSpec((T, D_OUT), lambda t, te: (t, 0)),
    )
    return pl.pallas_call(
        _mlp_body,
        grid_spec=grid_spec,
        out_shape=jax.ShapeDtypeStruct((P, D_OUT), jnp.float32),
        compiler_params=pltpu.CompilerParams(
            dimension_semantics=("arbitrary",)),
    )(tile_expert, x_sorted, eW1, eb1.reshape(N_EXPERTS, 1, H),
      eW2, eb2.reshape(N_EXPERTS, 1, H), eW3, eb3.reshape(N_EXPERTS, 1, D_OUT))


def _route(ids):
    oh = (ids[:, None] == jnp.arange(N_EXPERTS, dtype=jnp.int32)[None, :])
    rank = jnp.take_along_axis(jnp.cumsum(oh.astype(jnp.int32), axis=0) - 1,
                               ids[:, None], axis=1)[:, 0]
    counts = jnp.sum(oh.astype(jnp.int32), axis=0)
    tile_cnt = (counts + T - 1) // T
    cum_incl = jnp.cumsum(tile_cnt)
    pad_off = (cum_incl - tile_cnt) * T
    dest = pad_off[ids] + rank
    sort_idx = jnp.zeros((P,), jnp.int32).at[dest].set(
        jnp.arange(N_TOKENS, dtype=jnp.int32))
    t = jnp.arange(NT, dtype=jnp.int32)
    tile_expert = jnp.minimum(
        jnp.sum((t[:, None] >= cum_incl[None, :]).astype(jnp.int32), axis=1),
        N_EXPERTS - 1)
    return dest, sort_idx, tile_expert


def kernel(x, gW1, gb1, gW2, gb2, gW3, gb3, eW1, eb1, eW2, eb2, eW3, eb3):
    logits, cluster_ids = _gating(x, gW1, gb1, gW2, gb2, gW3, gb3)
    dest, sort_idx, tile_expert = _route(cluster_ids)
    x_sorted = _make_row_gather(P, D_IN)(x, sort_idx)
    y_sorted = _expert_mlp(tile_expert, x_sorted, eW1, eb1, eW2, eb2, eW3, eb3)
    outputs = _make_row_gather(N_TOKENS, D_OUT)(y_sorted, dest)
    return outputs, cluster_ids, logits


# speedup vs baseline: 1.2799x; 1.2799x over previous
import functools

import jax
import jax.numpy as jnp
from jax import lax
from jax.experimental import pallas as pl
from jax.experimental.pallas import tpu as pltpu
from jax.experimental.pallas import tpu_sc as plsc

N_TOKENS = 4096
D_IN = 1024
D_OUT = 1024
N_EXPERTS = 8
H = 1024

T = 128
P = N_TOKENS + N_EXPERTS * T
NT = P // T


def _gating_body(x_ref, w1_ref, b1_ref, w2_ref, b2_ref, w3_ref, b3_ref,
                 logits_ref, ids_ref):
    h = jnp.maximum(jnp.dot(x_ref[...], w1_ref[...],
                            preferred_element_type=jnp.float32) + b1_ref[...], 0.0)
    h = jnp.maximum(jnp.dot(h, w2_ref[...],
                            preferred_element_type=jnp.float32) + b2_ref[...], 0.0)
    lg = jnp.dot(h, w3_ref[...], preferred_element_type=jnp.float32) + b3_ref[...]
    logits_ref[...] = lg
    m = jnp.max(lg, axis=1, keepdims=True)
    cols = lax.broadcasted_iota(jnp.int32, lg.shape, 1)
    first_max = jnp.min(jnp.where(lg == m, cols, N_EXPERTS), axis=1, keepdims=True)
    ids_ref[...] = first_max.astype(jnp.int32)


def _gating(x, gW1, gb1, gW2, gb2, gW3, gb3):
    logits, ids = pl.pallas_call(
        _gating_body,
        out_shape=(
            jax.ShapeDtypeStruct((N_TOKENS, N_EXPERTS), jnp.float32),
            jax.ShapeDtypeStruct((N_TOKENS, 1), jnp.int32),
        ),
    )(x, gW1, gb1.reshape(1, -1), gW2, gb2.reshape(1, -1), gW3,
      gb3.reshape(1, -1))
    return logits, ids.reshape(N_TOKENS)


@functools.lru_cache(maxsize=None)
def _make_row_gather(n_rows_out, d, tag):
    info = plsc.get_sparse_core_info()
    nc, ns = info.num_cores, info.num_subcores
    nw = nc * ns
    per_w = n_rows_out // nw
    ch = 32
    n_ch = per_w // ch
    mesh = plsc.VectorSubcoreMesh(core_axis_name="c", subcore_axis_name="s")

    @functools.partial(
        pl.kernel,
        mesh=mesh,
        name=tag,
        out_type=jax.ShapeDtypeStruct((n_rows_out, d), jnp.float32),
        scratch_types=[
            pltpu.VMEM((n_ch, ch), jnp.int32),
            pltpu.VMEM((ch, d), jnp.float32),
            pltpu.VMEM((ch, d), jnp.float32),
            pltpu.SemaphoreType.DMA,
            pltpu.SemaphoreType.DMA,
            pltpu.SemaphoreType.DMA,
            pltpu.SemaphoreType.DMA,
        ],
    )
    def gather(table_hbm, idx_hbm, out_hbm, idx_v, buf0, buf1,
               gsem0, gsem1, osem0, osem1):
        wid = lax.axis_index("s") * nc + lax.axis_index("c")
        pltpu.sync_copy(idx_hbm.at[wid], idx_v)
        base = wid * per_w
        bufs, gsems, osems = (buf0, buf1), (gsem0, gsem1), (osem0, osem1)

        def start_gather(c):
            return pltpu.async_copy(table_hbm.at[idx_v.at[c]], bufs[c % 2],
                                    gsems[c % 2])

        def start_out(c):
            return pltpu.async_copy(bufs[c % 2],
                                    out_hbm.at[pl.ds(base + c * ch, ch)],
                                    osems[c % 2])

        cp = start_gather(0)
        outcp = [None, None]
        for c in range(n_ch):
            cp.wait()
            outcp[c % 2] = start_out(c)
            if c + 1 < n_ch:
                if outcp[(c + 1) % 2] is not None:
                    outcp[(c + 1) % 2].wait()
                cp = start_gather(c + 1)
        outcp[(n_ch - 1) % 2].wait()
        if n_ch >= 2:
            outcp[n_ch % 2].wait()

    def run(table, idx):
        return gather(table, idx.reshape(nw, n_ch, ch))

    return run


def _mlp_body(te_ref, xs_ref, w1_ref, b1_ref, w2_ref, b2_ref, w3_ref, b3_ref,
              o_ref):
    del te_ref
    h = jnp.maximum(jnp.dot(xs_ref[...], w1_ref[0],
                            preferred_element_type=jnp.float32) + b1_ref[0], 0.0)
    h = jnp.maximum(jnp.dot(h, w2_ref[0],
                            preferred_element_type=jnp.float32) + b2_ref[0], 0.0)
    o_ref[...] = jnp.dot(h, w3_ref[0],
                         preferred_element_type=jnp.float32) + b3_ref[0]


def _expert_mlp(tile_expert, x_sorted, eW1, eb1, eW2, eb2, eW3, eb3):
    grid_spec = pltpu.PrefetchScalarGridSpec(
        num_scalar_prefetch=1,
        grid=(NT,),
        in_specs=[
            pl.BlockSpec((T, D_IN), lambda t, te: (t, 0)),
            pl.BlockSpec((1, D_IN, H), lambda t, te: (te[t], 0, 0)),
            pl.BlockSpec((1, 1, H), lambda t, te: (te[t], 0, 0)),
            pl.BlockSpec((1, H, H), lambda t, te: (te[t], 0, 0)),
            pl.BlockSpec((1, 1, H), lambda t, te: (te[t], 0, 0)),
            pl.BlockSpec((1, H, D_OUT), lambda t, te: (te[t], 0, 0)),
            pl.BlockSpec((1, 1, D_OUT), lambda t, te: (te[t], 0, 0)),
        ],
        out_specs=pl.BlockSpec((T, D_OUT), lambda t, te: (t, 0)),
    )
    return pl.pallas_call(
        _mlp_body,
        grid_spec=grid_spec,
        out_shape=jax.ShapeDtypeStruct((P, D_OUT), jnp.float32),
        compiler_params=pltpu.CompilerParams(
            dimension_semantics=("arbitrary",)),
    )(tile_expert, x_sorted, eW1, eb1.reshape(N_EXPERTS, 1, H),
      eW2, eb2.reshape(N_EXPERTS, 1, H), eW3, eb3.reshape(N_EXPERTS, 1, D_OUT))


def _route(ids):
    oh = (ids[:, None] == jnp.arange(N_EXPERTS, dtype=jnp.int32)[None, :])
    rank = jnp.take_along_axis(jnp.cumsum(oh.astype(jnp.int32), axis=0) - 1,
                               ids[:, None], axis=1)[:, 0]
    counts = jnp.sum(oh.astype(jnp.int32), axis=0)
    tile_cnt = (counts + T - 1) // T
    cum_incl = jnp.cumsum(tile_cnt)
    pad_off = (cum_incl - tile_cnt) * T
    dest = pad_off[ids] + rank
    sort_idx = (jnp.arange(P, dtype=jnp.int32) % N_TOKENS).at[dest].set(
        jnp.arange(N_TOKENS, dtype=jnp.int32))
    t = jnp.arange(NT, dtype=jnp.int32)
    tile_expert = jnp.minimum(
        jnp.sum((t[:, None] >= cum_incl[None, :]).astype(jnp.int32), axis=1),
        N_EXPERTS - 1)
    return dest, sort_idx, tile_expert


def kernel(x, gW1, gb1, gW2, gb2, gW3, gb3, eW1, eb1, eW2, eb2, eW3, eb3):
    logits, cluster_ids = _gating(x, gW1, gb1, gW2, gb2, gW3, gb3)
    dest, sort_idx, tile_expert = _route(cluster_ids)
    x_sorted = _make_row_gather(P, D_IN, "dispatch_gather")(x, sort_idx)
    y_sorted = _expert_mlp(tile_expert, x_sorted, eW1, eb1, eW2, eb2, eW3, eb3)
    outputs = _make_row_gather(N_TOKENS, D_OUT, "undispatch_gather")(y_sorted, dest)
    return outputs, cluster_ids, logits


# speedup vs baseline: 1.6062x; 1.2550x over previous
import functools

import jax
import jax.numpy as jnp
from jax import lax
from jax.experimental import pallas as pl
from jax.experimental.pallas import tpu as pltpu
from jax.experimental.pallas import tpu_sc as plsc

N_TOKENS = 4096
D_IN = 1024
D_OUT = 1024
N_EXPERTS = 8
H = 1024

T = 128
P = N_TOKENS + N_EXPERTS * T
NT = P // T


def _gating_body(x_ref, w1_ref, b1_ref, w2_ref, b2_ref, w3_ref, b3_ref,
                 logits_ref, ids_ref, dest_ref, ci_ref):
    h = jnp.maximum(jnp.dot(x_ref[...], w1_ref[...],
                            preferred_element_type=jnp.float32) + b1_ref[...], 0.0)
    h = jnp.maximum(jnp.dot(h, w2_ref[...],
                            preferred_element_type=jnp.float32) + b2_ref[...], 0.0)
    lg = jnp.dot(h, w3_ref[...], preferred_element_type=jnp.float32) + b3_ref[...]
    logits_ref[...] = lg

    m = jnp.max(lg, axis=1, keepdims=True)
    cols = lax.broadcasted_iota(jnp.int32, lg.shape, 1)
    ids = jnp.min(jnp.where(lg == m, cols, N_EXPERTS), axis=1, keepdims=True)
    ids_ref[...] = ids

    oh = (cols == ids).astype(jnp.int32)
    cs = oh
    sh = 1
    while sh < N_TOKENS:
        cs = cs + jnp.concatenate(
            [jnp.zeros((sh, N_EXPERTS), jnp.int32), cs[:-sh, :]], axis=0)
        sh *= 2
    counts = cs[N_TOKENS - 1:N_TOKENS, :]
    tile_cnt = (counts + (T - 1)) // T
    ci = tile_cnt
    sh = 1
    while sh < N_EXPERTS:
        ci = ci + jnp.concatenate(
            [jnp.zeros((1, sh), jnp.int32), ci[:, :-sh]], axis=1)
        sh *= 2
    ci_ref[...] = ci
    pad_off = (ci - tile_cnt) * T
    rank = jnp.sum(oh * (cs - 1), axis=1, keepdims=True)
    base = jnp.sum(oh * pad_off, axis=1, keepdims=True)
    dest_ref[...] = base + rank


def _gating(x, gW1, gb1, gW2, gb2, gW3, gb3):
    logits, ids, dest, ci = pl.pallas_call(
        _gating_body,
        out_shape=(
            jax.ShapeDtypeStruct((N_TOKENS, N_EXPERTS), jnp.float32),
            jax.ShapeDtypeStruct((N_TOKENS, 1), jnp.int32),
            jax.ShapeDtypeStruct((N_TOKENS, 1), jnp.int32),
            jax.ShapeDtypeStruct((1, N_EXPERTS), jnp.int32),
        ),
    )(x, gW1, gb1.reshape(1, -1), gW2, gb2.reshape(1, -1), gW3,
      gb3.reshape(1, -1))
    return logits, ids.reshape(N_TOKENS), dest.reshape(N_TOKENS), ci.reshape(N_EXPERTS)


@functools.lru_cache(maxsize=None)
def _make_row_mover(n_rows, n_rows_store, d, tag, scatter):
    info = plsc.get_sparse_core_info()
    nc, ns = info.num_cores, info.num_subcores
    nw = nc * ns
    per_w = n_rows // nw
    ch = 32
    n_ch = per_w // ch
    mesh = plsc.VectorSubcoreMesh(core_axis_name="c", subcore_axis_name="s")

    @functools.partial(
        pl.kernel,
        mesh=mesh,
        name=tag,
        out_type=jax.ShapeDtypeStruct((n_rows_store, d), jnp.float32),
        scratch_types=[
            pltpu.VMEM((n_ch, ch), jnp.int32),
            pltpu.VMEM((ch, d), jnp.float32),
            pltpu.VMEM((ch, d), jnp.float32),
            pltpu.SemaphoreType.DMA,
            pltpu.SemaphoreType.DMA,
            pltpu.SemaphoreType.DMA,
            pltpu.SemaphoreType.DMA,
        ],
    )
    def mover(table_hbm, idx_hbm, out_hbm, idx_v, buf0, buf1,
              gsem0, gsem1, osem0, osem1):
        wid = lax.axis_index("s") * nc + lax.axis_index("c")
        pltpu.sync_copy(idx_hbm.at[wid], idx_v)
        base = wid * per_w
        bufs, gsems, osems = (buf0, buf1), (gsem0, gsem1), (osem0, osem1)

        def start_in(c):
            if scatter:
                src = table_hbm.at[pl.ds(base + c * ch, ch)]
            else:
                src = table_hbm.at[idx_v.at[c]]
            return pltpu.async_copy(src, bufs[c % 2], gsems[c % 2])

        def start_out(c):
            if scatter:
                dst = out_hbm.at[idx_v.at[c]]
            else:
                dst = out_hbm.at[pl.ds(base + c * ch, ch)]
            return pltpu.async_copy(bufs[c % 2], dst, osems[c % 2])

        cp = start_in(0)
        outcp = [None, None]
        for c in range(n_ch):
            cp.wait()
            outcp[c % 2] = start_out(c)
            if c + 1 < n_ch:
                if outcp[(c + 1) % 2] is not None:
                    outcp[(c + 1) % 2].wait()
                cp = start_in(c + 1)
        outcp[(n_ch - 1) % 2].wait()
        if n_ch >= 2:
            outcp[n_ch % 2].wait()

    def run(table, idx):
        return mover(table, idx.reshape(nw, n_ch, ch))

    return run


def _expert_of(t, ci):
    e = jnp.int32(0)
    for k in range(N_EXPERTS):
        e = e + jnp.where(t >= ci[k], 1, 0).astype(jnp.int32)
    return jnp.minimum(e, N_EXPERTS - 1)


def _mlp_body(ci_ref, xs_ref, w1_ref, b1_ref, w2_ref, b2_ref, w3_ref, b3_ref,
              o_ref):
    del ci_ref
    h = jnp.maximum(jnp.dot(xs_ref[...], w1_ref[0],
                            preferred_element_type=jnp.float32) + b1_ref[0], 0.0)
    h = jnp.maximum(jnp.dot(h, w2_ref[0],
                            preferred_element_type=jnp.float32) + b2_ref[0], 0.0)
    o_ref[...] = jnp.dot(h, w3_ref[0],
                         preferred_element_type=jnp.float32) + b3_ref[0]


def _expert_mlp(ci, x_sorted, eW1, eb1, eW2, eb2, eW3, eb3):
    def wmap(t, ci_ref):
        return (_expert_of(t, ci_ref), 0, 0)

    grid_spec = pltpu.PrefetchScalarGridSpec(
        num_scalar_prefetch=1,
        grid=(NT,),
        in_specs=[
            pl.BlockSpec((T, D_IN), lambda t, ci_ref: (t, 0)),
            pl.BlockSpec((1, D_IN, H), wmap),
            pl.BlockSpec((1, 1, H), wmap),
            pl.BlockSpec((1, H, H), wmap),
            pl.BlockSpec((1, 1, H), wmap),
            pl.BlockSpec((1, H, D_OUT), wmap),
            pl.BlockSpec((1, 1, D_OUT), wmap),
        ],
        out_specs=pl.BlockSpec((T, D_OUT), lambda t, ci_ref: (t, 0)),
    )
    return pl.pallas_call(
        _mlp_body,
        grid_spec=grid_spec,
        out_shape=jax.ShapeDtypeStruct((P, D_OUT), jnp.float32),
        compiler_params=pltpu.CompilerParams(
            dimension_semantics=("arbitrary",)),
    )(ci, x_sorted, eW1, eb1.reshape(N_EXPERTS, 1, H),
      eW2, eb2.reshape(N_EXPERTS, 1, H), eW3, eb3.reshape(N_EXPERTS, 1, D_OUT))


def kernel(x, gW1, gb1, gW2, gb2, gW3, gb3, eW1, eb1, eW2, eb2, eW3, eb3):
    logits, cluster_ids, dest, ci = _gating(x, gW1, gb1, gW2, gb2, gW3, gb3)
    x_sorted = _make_row_mover(N_TOKENS, P, D_IN, "dispatch_scatter", True)(x, dest)
    y_sorted = _expert_mlp(ci, x_sorted, eW1, eb1, eW2, eb2, eW3, eb3)
    outputs = _make_row_mover(N_TOKENS, N_TOKENS, D_OUT, "undispatch_gather",
                              False)(y_sorted, dest)
    return outputs, cluster_ids, logits
